# baseline (speedup 1.0000x reference)
import jax
import jax.numpy as jnp
from jax.experimental import pallas as pl
from jax.experimental.pallas import tpu as pltpu

NBINS = 512
MIN_VAL = -2.0
MAX_VAL = 2.0
EPS = 1e-6
THRESH = 0.008
BW = (MAX_VAL - MIN_VAL) / (NBINS - 1)
TDOT = 1.0 + EPS - (THRESH * THRESH) / 2.0

N = 4096
BLK = 1024
NB = N // BLK
LANES = 128
NF = BLK // LANES


def _dist(d):
    return jnp.sqrt(jnp.maximum((1.0 - d + EPS) * 2.0, 0.0))


def _fold_rowmax(dm):
    acc = dm[:, 0:LANES]
    for k in range(1, NF):
        acc = jnp.maximum(acc, dm[:, k * LANES:(k + 1) * LANES])
    return acc


def _loss_kernel(a_ref, p_ref, loss_ref, posd_ref, rowmax_ref, colmax_ref):
    i = pl.program_id(0)
    j = pl.program_id(1)

    dot = jax.lax.dot_general(
        a_ref[...], p_ref[...], (((1,), (1,)), ((), ())),
        preferred_element_type=jnp.float32)

    masked = jnp.where(dot > TDOT, -2.0, dot)

    @pl.when(i == j)
    def _():
        r = jax.lax.broadcasted_iota(jnp.int32, (BLK, BLK), 0)
        c = jax.lax.broadcasted_iota(jnp.int32, (BLK, BLK), 1)
        diag = r == c
        dm = jnp.where(diag, -2.0, masked)
        rm2 = _fold_rowmax(dm)
        cm = jnp.max(dm, axis=0)
        pd2 = _fold_rowmax(jnp.where(diag, dot, -3.0))
        posd_ref[pl.ds(i * BLK, BLK), :] = pd2

        @pl.when(j == 0)
        def _():
            rowmax_ref[pl.ds(i * BLK, BLK), :] = rm2

        @pl.when(j > 0)
        def _():
            rowmax_ref[pl.ds(i * BLK, BLK), :] = jnp.maximum(
                rowmax_ref[pl.ds(i * BLK, BLK), :], rm2)

        @pl.when(i == 0)
        def _():
            colmax_ref[:, pl.ds(j * BLK, BLK)] = cm[None, :]

        @pl.when(i > 0)
        def _():
            colmax_ref[:, pl.ds(j * BLK, BLK)] = jnp.maximum(
                colmax_ref[:, pl.ds(j * BLK, BLK)], cm[None, :])

    @pl.when(i != j)
    def _():
        rm2 = _fold_rowmax(masked)
        cm = jnp.max(masked, axis=0)

        @pl.when(j == 0)
        def _():
            rowmax_ref[pl.ds(i * BLK, BLK), :] = rm2

        @pl.when(j > 0)
        def _():
            rowmax_ref[pl.ds(i * BLK, BLK), :] = jnp.maximum(
                rowmax_ref[pl.ds(i * BLK, BLK), :], rm2)

        @pl.when(i == 0)
        def _():
            colmax_ref[:, pl.ds(j * BLK, BLK)] = cm[None, :]

        @pl.when(i > 0)
        def _():
            colmax_ref[:, pl.ds(j * BLK, BLK)] = jnp.maximum(
                colmax_ref[:, pl.ds(j * BLK, BLK)], cm[None, :])

    @pl.when(jnp.logical_and(i == NB - 1, j == NB - 1))
    def _():
        posd = jnp.max(posd_ref[...], axis=1)
        rowm = jnp.max(rowmax_ref[...], axis=1)
        pos = _dist(posd)
        neg = _dist(jnp.maximum(rowm, colmax_ref[...][0]))
        hv = pos - neg

        lo = jnp.floor((hv - MIN_VAL) / BW).astype(jnp.int32)
        alpha = 1.0 - (hv - MIN_VAL - lo.astype(jnp.float32) * BW) / BW
        hi = jnp.clip(lo + 1, 0, NBINS - 1)
        lo_w = jnp.where(lo < 0, lo + NBINS, lo)
        lo_ok = jnp.logical_and(lo_w >= 0, lo_w <= NBINS - 1)
        a_lo = jnp.where(lo_ok, alpha, 0.0)
        a_hi = 1.0 - alpha

        biota = jax.lax.broadcasted_iota(jnp.int32, (NBINS, N), 0)
        lo_b = jnp.broadcast_to(lo_w[None, :], (NBINS, N))
        hi_b = jnp.broadcast_to(hi[None, :], (NBINS, N))
        oh_lo = (biota == lo_b).astype(jnp.float32)
        oh_hi = (biota == hi_b).astype(jnp.float32)
        le_lo = (biota <= lo_b).astype(jnp.float32)

        hist = (jax.lax.dot_general(
                    oh_lo, a_lo.reshape(N, 1), (((1,), (0,)), ((), ())),
                    preferred_element_type=jnp.float32)
                + jax.lax.dot_general(
                    oh_hi, a_hi.reshape(N, 1), (((1,), (0,)), ((), ())),
                    preferred_element_type=jnp.float32))

        s1 = jnp.sum(hist)
        h1 = hist / (s1 + 1e-6)
        s2 = jnp.sum(h1)
        pdf = h1 / s2

        w = jax.lax.dot_general(
            pdf.reshape(1, NBINS), le_lo, (((1,), (0,)), ((), ())),
            preferred_element_type=jnp.float32)

        diff = (pos - neg).reshape(N, 1)
        loss = jax.lax.dot_general(
            w, diff, (((1,), (0,)), ((), ())),
            preferred_element_type=jnp.float32) / N
        loss_ref[...] = loss


def kernel(x, histogram):
    del histogram
    a = x[:N, :]
    p = x[N:, :]
    out = pl.pallas_call(
        _loss_kernel,
        grid=(NB, NB),
        in_specs=[
            pl.BlockSpec((BLK, 128), lambda i, j: (i, 0)),
            pl.BlockSpec((BLK, 128), lambda i, j: (j, 0)),
        ],
        out_specs=pl.BlockSpec((1, 1), lambda i, j: (0, 0)),
        out_shape=jax.ShapeDtypeStruct((1, 1), jnp.float32),
        scratch_shapes=[
            pltpu.VMEM((N, LANES), jnp.float32),
            pltpu.VMEM((N, LANES), jnp.float32),
            pltpu.VMEM((1, N), jnp.float32),
        ],
    )(a, p)
    return out[0, 0]
